# baseline (speedup 1.0000x reference)
import functools

import jax
import jax.numpy as jnp
from jax import lax
from jax.experimental import pallas as pl
from jax.experimental.pallas import tpu as pltpu
from jax.experimental.pallas import tpu_sc as plsc

_PW = 48


def _matmul_body(w_ref, x_ref, out_ref):
    out_ref[...] = lax.dot_general(
        w_ref[...], x_ref[...],
        dimension_numbers=(((1,), (1,)), ((), ())),
        preferred_element_type=jnp.float32,
    )


def _sc_key(f):
    bu = lax.bitcast_convert_type(f, jnp.uint32)
    key = jnp.where((bu >> 31) == 0, bu | jnp.uint32(0x80000000), ~bu)
    return jnp.where(f == 0.0, jnp.uint32(0x80000000), key)


def _sc_thresh_body(logits, params, row_v, keys_v, hist_v, tbuf_v, pbuf_v,
                    *, E, T, k, TB):
    c = lax.axis_index("c")
    s = lax.axis_index("s")
    NV = T // 16
    NVB = TB // 16
    NB = T // TB
    lane = lax.iota(jnp.int32, 16)
    lane_base = lane * 256
    ones16 = jnp.ones((16,), jnp.int32)

    pltpu.sync_copy(logits.at[s], row_v)

    def zero_hist():
        def zh(j, _):
            hist_v[pl.ds(j * 16, 16)] = jnp.zeros((16,), jnp.int32)
            return 0
        lax.fori_loop(0, (16 * 256) // 16, zh, 0)

    def fold_and_pick(k_rem):
        def fc(ci, _):
            def fl(l, a):
                return a + hist_v[pl.ds(l * 256 + ci * 16, 16)]
            acc = lax.fori_loop(0, 16, fl, jnp.zeros((16,), jnp.int32))
            plsc.store_scatter(tbuf_v, [lane * 16 + ci], acc)
            return 0
        lax.fori_loop(0, 16, fc, 0)

        def gsum(j, a):
            return a + tbuf_v[pl.ds(j * 16, 16)]
        g = lax.fori_loop(0, 16, gsum, jnp.zeros((16,), jnp.int32))
        sg = lax.rev(plsc.cumsum(lax.rev(g, (0,))), (0,)) - g
        mg = (sg < k_rem) & (sg + g >= k_rem)
        ci = jnp.sum(jnp.where(mg, lane, 0))
        s_chunk = jnp.sum(jnp.where(mg, sg, 0))
        t = plsc.load_gather(tbuf_v, [lane * 16 + ci])
        st = lax.rev(plsc.cumsum(lax.rev(t, (0,))), (0,)) - t + s_chunk
        mt = (st < k_rem) & (st + t >= k_rem)
        j = jnp.sum(jnp.where(mt, lane, 0))
        s_above = jnp.sum(jnp.where(mt, st, 0))
        return ci * 16 + j, k_rem - s_above

    zero_hist()

    def r1(i, _):
        f = row_v[pl.ds(i * 16, 16)]
        key = _sc_key(f)
        keys_v[pl.ds(i * 16, 16)] = key
        d = ((key >> 24) & 255).astype(jnp.int32)
        plsc.addupdate_scatter(hist_v, [lane_base + d], ones16)
        return 0
    lax.fori_loop(0, NV, r1, 0)
    bin0, k_rem = fold_and_pick(jnp.int32(k))
    prefix = bin0.astype(jnp.uint32) << 24

    for shift in (16, 8, 0):
        zero_hist()
        hs = shift + 8

        def rb(i, _, hs=hs, shift=shift, prefix=prefix):
            key = keys_v[pl.ds(i * 16, 16)]
            act = (key >> hs) == (prefix >> hs)
            d = ((key >> shift) & 255).astype(jnp.int32)
            plsc.addupdate_scatter(hist_v, [lane_base + d], ones16, mask=act)
            return 0
        lax.fori_loop(0, NV, rb, 0)
        bin_, k_rem = fold_and_pick(k_rem)
        prefix = prefix | (bin_.astype(jnp.uint32) << shift)

    tkey = prefix
    allowed = k_rem

    tki = lax.bitcast_convert_type(jnp.zeros((16,), jnp.uint32) + tkey,
                                   jnp.int32)
    plsc.store_scatter(
        pbuf_v, [lane],
        jnp.where(lane == 0, tki, allowed), mask=lane < 2)

    def tb(blk, run):
        plsc.store_scatter(
            pbuf_v, [jnp.full((16,), 2, jnp.int32) + blk],
            jnp.full((16,), 1, jnp.int32) * run, mask=lane == 0)

        def ti(i, r):
            key = keys_v[pl.ds((blk * NVB + i) * 16, 16)]
            return r + jnp.sum((key == tkey).astype(jnp.int32))
        return lax.fori_loop(0, NVB, ti, run)
    lax.fori_loop(0, NB, tb, jnp.int32(0))

    @pl.when(c == 0)
    def _():
        pltpu.sync_copy(pbuf_v, params.at[s])


def _sc_assign_body(logits, params, eout, wout,
                    col_v, ptab_v, oute_v, outw_v, sem,
                    *, E, T, k, TB):
    c = lax.axis_index("c")
    s = lax.axis_index("s")
    wid = c * 16 + s
    base = wid * TB
    NVB = TB // 16
    lane = lax.iota(jnp.int32, 16)

    col_cp = pltpu.async_copy(logits.at[:, pl.ds(base, TB)], col_v, sem)
    pltpu.sync_copy(params, ptab_v)
    col_cp.wait()

    def _lane0(v):
        return jnp.sum(jnp.where(lane == 0, v, 0))

    te, alw, bpre = [], [], []
    widv = jnp.zeros((16,), jnp.int32) + (2 + wid)
    for e in range(E):
        row = ptab_v[e, pl.ds(0, 16)]
        rowu = lax.bitcast_convert_type(row, jnp.uint32)
        te.append(jnp.sum(jnp.where(lane == 0, rowu, jnp.uint32(0))))
        alw.append(jnp.sum(jnp.where(lane == 1, row, 0)))
        bpre.append(_lane0(plsc.load_gather(
            ptab_v, [jnp.full((16,), e, jnp.int32), widv])))
    minus_inf = jnp.full((16,), -jnp.inf, jnp.float32)

    def p3(i, runs):
        ls = [col_v[e, pl.ds(i * 16, 16)] for e in range(E)]
        m = minus_inf
        for e in range(E):
            m = jnp.maximum(m, ls[e])
        den = jnp.zeros((16,), jnp.float32)
        for e in range(E):
            den = den + jnp.exp(ls[e] - m)
        estar = jnp.full((16,), -1, jnp.int32)
        val = jnp.zeros((16,), jnp.float32)
        new_runs = []
        for e in range(E):
            f = ls[e]
            key = _sc_key(f)
            eqm = key == te[e]
            ic = plsc.cumsum(eqm.astype(jnp.int32))
            sel = (key > te[e]) | (eqm & ((bpre[e] + runs[e] + ic) <= alw[e]))
            estar = jnp.where(sel, jnp.int32(e), estar)
            val = jnp.where(sel, f, val)
            new_runs.append(runs[e] + jnp.sum(jnp.where(lane == 15, ic, 0)))
        picked = estar >= 0
        w = jnp.exp(val - m) / den
        oute_v[pl.ds(i * 16, 16)] = jnp.maximum(estar, jnp.int32(0))
        outw_v[pl.ds(i * 16, 16)] = jnp.where(picked, w, jnp.float32(1.0))
        return tuple(new_runs)
    lax.fori_loop(0, NVB, p3, tuple(jnp.int32(0) for _ in range(E)))

    pltpu.sync_copy(oute_v, eout.at[pl.ds(base, TB)])
    pltpu.sync_copy(outw_v, wout.at[pl.ds(base, TB)])


def _sc_select(logits, *, E, T, k):
    NB = 32
    TB = T // NB
    mesh = plsc.VectorSubcoreMesh(core_axis_name="c", subcore_axis_name="s")
    cp = pltpu.CompilerParams(needs_layout_passes=False)

    thresh = pl.kernel(
        functools.partial(_sc_thresh_body, E=E, T=T, k=k, TB=TB),
        out_type=jax.ShapeDtypeStruct((E, _PW), jnp.int32),
        mesh=mesh,
        scratch_types=[
            pltpu.VMEM((T,), jnp.float32),
            pltpu.VMEM((T,), jnp.uint32),
            pltpu.VMEM((16 * 256,), jnp.int32),
            pltpu.VMEM((256,), jnp.int32),
            pltpu.VMEM((_PW,), jnp.int32),
        ],
        compiler_params=cp,
    )
    params = thresh(logits)

    assign = pl.kernel(
        functools.partial(_sc_assign_body, E=E, T=T, k=k, TB=TB),
        out_type=(
            jax.ShapeDtypeStruct((T,), jnp.int32),
            jax.ShapeDtypeStruct((T,), jnp.float32),
        ),
        mesh=mesh,
        scratch_types=[
            pltpu.VMEM((E, TB), jnp.float32),
            pltpu.VMEM((E, _PW), jnp.int32),
            pltpu.VMEM((TB,), jnp.int32),
            pltpu.VMEM((TB,), jnp.float32),
            pltpu.SemaphoreType.DMA,
        ],
        compiler_params=cp,
    )
    return assign(logits, params)


def kernel(x, W_gate):
    B, S, D = x.shape
    E = W_gate.shape[0]
    T = B * S
    k = min(int(T / E * 1.0), T)

    xf = x.reshape(T, D)
    TBLK = 512
    logits = pl.pallas_call(
        _matmul_body,
        grid=(T // TBLK,),
        in_specs=[
            pl.BlockSpec((E, D), lambda i: (0, 0)),
            pl.BlockSpec((TBLK, D), lambda i: (i, 0)),
        ],
        out_specs=pl.BlockSpec((E, TBLK), lambda i: (0, i)),
        out_shape=jax.ShapeDtypeStruct((E, T), jnp.float32),
    )(W_gate, xf)

    expert, weight = _sc_select(logits, E=E, T=T, k=k)

    expert_out = expert.reshape(B, S, 1)
    weight_out = weight.reshape(B, S, 1).astype(x.dtype)
    aux_loss = jnp.array(0.0, dtype=x.dtype)
    return (expert_out, weight_out, aux_loss)


# speedup vs baseline: 1.0911x; 1.0911x over previous
import functools

import jax
import jax.numpy as jnp
from jax import lax
from jax.experimental import pallas as pl
from jax.experimental.pallas import tpu as pltpu
from jax.experimental.pallas import tpu_sc as plsc

_PW = 48


def _matmul_body(w_ref, x_ref, out_ref):
    out_ref[...] = lax.dot_general(
        w_ref[...], x_ref[...],
        dimension_numbers=(((1,), (1,)), ((), ())),
        preferred_element_type=jnp.float32,
    )


def _sc_key(f):
    bu = lax.bitcast_convert_type(f, jnp.uint32)
    key = jnp.where((bu >> 31) == 0, bu | jnp.uint32(0x80000000), ~bu)
    return jnp.where(f == 0.0, jnp.uint32(0x80000000), key)


def _sc_thresh_body(logits, params, row_v, keys_v, hist_v, tbuf_v, pbuf_v,
                    *, E, T, k, TB):
    c = lax.axis_index("c")
    s = lax.axis_index("s")
    NV = T // 16
    NVB = TB // 16
    NB = T // TB
    lane = lax.iota(jnp.int32, 16)
    lane_base = lane * 256
    ones16 = jnp.ones((16,), jnp.int32)

    pltpu.sync_copy(logits.at[s], row_v)

    def zero_hist():
        def zh(j, _):
            hist_v[pl.ds(j * 16, 16)] = jnp.zeros((16,), jnp.int32)
            return 0
        lax.fori_loop(0, (16 * 256) // 16, zh, 0, unroll=8)

    def fold_and_pick(k_rem):
        def fc(ci, _):
            def fl(l, a):
                return a + hist_v[pl.ds(l * 256 + ci * 16, 16)]
            acc = lax.fori_loop(0, 16, fl, jnp.zeros((16,), jnp.int32), unroll=4)
            plsc.store_scatter(tbuf_v, [lane * 16 + ci], acc)
            return 0
        lax.fori_loop(0, 16, fc, 0)

        def gsum(j, a):
            return a + tbuf_v[pl.ds(j * 16, 16)]
        g = lax.fori_loop(0, 16, gsum, jnp.zeros((16,), jnp.int32))
        sg = lax.rev(plsc.cumsum(lax.rev(g, (0,))), (0,)) - g
        mg = (sg < k_rem) & (sg + g >= k_rem)
        ci = jnp.sum(jnp.where(mg, lane, 0))
        s_chunk = jnp.sum(jnp.where(mg, sg, 0))
        t = plsc.load_gather(tbuf_v, [lane * 16 + ci])
        st = lax.rev(plsc.cumsum(lax.rev(t, (0,))), (0,)) - t + s_chunk
        mt = (st < k_rem) & (st + t >= k_rem)
        j = jnp.sum(jnp.where(mt, lane, 0))
        s_above = jnp.sum(jnp.where(mt, st, 0))
        return ci * 16 + j, k_rem - s_above

    zero_hist()

    def r1(i, _):
        f = row_v[pl.ds(i * 16, 16)]
        key = _sc_key(f)
        keys_v[pl.ds(i * 16, 16)] = key
        d = ((key >> 24) & 255).astype(jnp.int32)
        plsc.addupdate_scatter(hist_v, [lane_base + d], ones16)
        return 0
    lax.fori_loop(0, NV, r1, 0, unroll=8)
    bin0, k_rem = fold_and_pick(jnp.int32(k))
    prefix = bin0.astype(jnp.uint32) << 24

    for shift in (16, 8, 0):
        zero_hist()
        hs = shift + 8

        def rb(i, _, hs=hs, shift=shift, prefix=prefix):
            key = keys_v[pl.ds(i * 16, 16)]
            act = (key >> hs) == (prefix >> hs)
            d = ((key >> shift) & 255).astype(jnp.int32)
            plsc.addupdate_scatter(hist_v, [lane_base + d], ones16, mask=act)
            return 0
        lax.fori_loop(0, NV, rb, 0, unroll=8)
        bin_, k_rem = fold_and_pick(k_rem)
        prefix = prefix | (bin_.astype(jnp.uint32) << shift)

    tkey = prefix
    allowed = k_rem

    tki = lax.bitcast_convert_type(jnp.zeros((16,), jnp.uint32) + tkey,
                                   jnp.int32)
    plsc.store_scatter(
        pbuf_v, [lane],
        jnp.where(lane == 0, tki, allowed), mask=lane < 2)

    def tb(blk, run):
        plsc.store_scatter(
            pbuf_v, [jnp.full((16,), 2, jnp.int32) + blk],
            jnp.full((16,), 1, jnp.int32) * run, mask=lane == 0)

        def ti(i, r):
            key = keys_v[pl.ds((blk * NVB + i) * 16, 16)]
            return r + jnp.sum((key == tkey).astype(jnp.int32))
        return lax.fori_loop(0, NVB, ti, run, unroll=8)
    lax.fori_loop(0, NB, tb, jnp.int32(0))

    @pl.when(c == 0)
    def _():
        pltpu.sync_copy(pbuf_v, params.at[s])


def _sc_assign_body(logits, params, eout, wout,
                    col_v, ptab_v, oute_v, outw_v, sem,
                    *, E, T, k, TB):
    c = lax.axis_index("c")
    s = lax.axis_index("s")
    wid = c * 16 + s
    base = wid * TB
    NVB = TB // 16
    lane = lax.iota(jnp.int32, 16)

    col_cp = pltpu.async_copy(logits.at[:, pl.ds(base, TB)], col_v, sem)
    pltpu.sync_copy(params, ptab_v)
    col_cp.wait()

    def _lane0(v):
        return jnp.sum(jnp.where(lane == 0, v, 0))

    te, alw, bpre = [], [], []
    widv = jnp.zeros((16,), jnp.int32) + (2 + wid)
    for e in range(E):
        row = ptab_v[e, pl.ds(0, 16)]
        rowu = lax.bitcast_convert_type(row, jnp.uint32)
        te.append(jnp.sum(jnp.where(lane == 0, rowu, jnp.uint32(0))))
        alw.append(jnp.sum(jnp.where(lane == 1, row, 0)))
        bpre.append(_lane0(plsc.load_gather(
            ptab_v, [jnp.full((16,), e, jnp.int32), widv])))
    minus_inf = jnp.full((16,), -jnp.inf, jnp.float32)

    def p3(i, runs):
        ls = [col_v[e, pl.ds(i * 16, 16)] for e in range(E)]
        m = minus_inf
        for e in range(E):
            m = jnp.maximum(m, ls[e])
        den = jnp.zeros((16,), jnp.float32)
        for e in range(E):
            den = den + jnp.exp(ls[e] - m)
        estar = jnp.full((16,), -1, jnp.int32)
        val = jnp.zeros((16,), jnp.float32)
        new_runs = []
        for e in range(E):
            f = ls[e]
            key = _sc_key(f)
            eqm = key == te[e]
            ic = plsc.cumsum(eqm.astype(jnp.int32))
            sel = (key > te[e]) | (eqm & ((bpre[e] + runs[e] + ic) <= alw[e]))
            estar = jnp.where(sel, jnp.int32(e), estar)
            val = jnp.where(sel, f, val)
            new_runs.append(runs[e] + jnp.sum(jnp.where(lane == 15, ic, 0)))
        picked = estar >= 0
        w = jnp.exp(val - m) / den
        oute_v[pl.ds(i * 16, 16)] = jnp.maximum(estar, jnp.int32(0))
        outw_v[pl.ds(i * 16, 16)] = jnp.where(picked, w, jnp.float32(1.0))
        return tuple(new_runs)
    lax.fori_loop(0, NVB, p3, tuple(jnp.int32(0) for _ in range(E)))

    pltpu.sync_copy(oute_v, eout.at[pl.ds(base, TB)])
    pltpu.sync_copy(outw_v, wout.at[pl.ds(base, TB)])


def _sc_select(logits, *, E, T, k):
    NB = 32
    TB = T // NB
    mesh = plsc.VectorSubcoreMesh(core_axis_name="c", subcore_axis_name="s")
    cp = pltpu.CompilerParams(needs_layout_passes=False)

    thresh = pl.kernel(
        functools.partial(_sc_thresh_body, E=E, T=T, k=k, TB=TB),
        out_type=jax.ShapeDtypeStruct((E, _PW), jnp.int32),
        mesh=mesh,
        scratch_types=[
            pltpu.VMEM((T,), jnp.float32),
            pltpu.VMEM((T,), jnp.uint32),
            pltpu.VMEM((16 * 256,), jnp.int32),
            pltpu.VMEM((256,), jnp.int32),
            pltpu.VMEM((_PW,), jnp.int32),
        ],
        compiler_params=cp,
    )
    params = thresh(logits)

    assign = pl.kernel(
        functools.partial(_sc_assign_body, E=E, T=T, k=k, TB=TB),
        out_type=(
            jax.ShapeDtypeStruct((T,), jnp.int32),
            jax.ShapeDtypeStruct((T,), jnp.float32),
        ),
        mesh=mesh,
        scratch_types=[
            pltpu.VMEM((E, TB), jnp.float32),
            pltpu.VMEM((E, _PW), jnp.int32),
            pltpu.VMEM((TB,), jnp.int32),
            pltpu.VMEM((TB,), jnp.float32),
            pltpu.SemaphoreType.DMA,
        ],
        compiler_params=cp,
    )
    return assign(logits, params)


def kernel(x, W_gate):
    B, S, D = x.shape
    E = W_gate.shape[0]
    T = B * S
    k = min(int(T / E * 1.0), T)

    xf = x.reshape(T, D)
    TBLK = 512
    logits = pl.pallas_call(
        _matmul_body,
        grid=(T // TBLK,),
        in_specs=[
            pl.BlockSpec((E, D), lambda i: (0, 0)),
            pl.BlockSpec((TBLK, D), lambda i: (i, 0)),
        ],
        out_specs=pl.BlockSpec((E, TBLK), lambda i: (0, i)),
        out_shape=jax.ShapeDtypeStruct((E, T), jnp.float32),
    )(W_gate, xf)

    expert, weight = _sc_select(logits, E=E, T=T, k=k)

    expert_out = expert.reshape(B, S, 1)
    weight_out = weight.reshape(B, S, 1).astype(x.dtype)
    aux_loss = jnp.array(0.0, dtype=x.dtype)
    return (expert_out, weight_out, aux_loss)


# speedup vs baseline: 1.1242x; 1.0303x over previous
import functools

import jax
import jax.numpy as jnp
from jax import lax
from jax.experimental import pallas as pl
from jax.experimental.pallas import tpu as pltpu
from jax.experimental.pallas import tpu_sc as plsc

_PW = 48


def _matmul_body(w_ref, x_ref, out_ref):
    out_ref[...] = lax.dot_general(
        w_ref[...], x_ref[...],
        dimension_numbers=(((1,), (1,)), ((), ())),
        preferred_element_type=jnp.float32,
    )


def _sc_key(f):
    bu = lax.bitcast_convert_type(f, jnp.uint32)
    key = jnp.where((bu >> 31) == 0, bu | jnp.uint32(0x80000000), ~bu)
    return jnp.where(f == 0.0, jnp.uint32(0x80000000), key)


def _sc_thresh_body(logits, params, row_v, keys_v, hist_v, tbuf_v, pbuf_v,
                    ckey_v, cidx_v, *, E, T, k, TB):
    c = lax.axis_index("c")
    s = lax.axis_index("s")
    NV = T // 16
    NVB = TB // 16
    NB = T // TB
    lane = lax.iota(jnp.int32, 16)
    lane_base = lane * 256
    ones16 = jnp.ones((16,), jnp.int32)

    pltpu.sync_copy(logits.at[s], row_v)

    def zero_hist():
        def zh(j, _):
            hist_v[pl.ds(j * 16, 16)] = jnp.zeros((16,), jnp.int32)
            return 0
        lax.fori_loop(0, (16 * 256) // 16, zh, 0, unroll=8)

    def fold_and_pick(k_rem):
        def fc(ci, _):
            def fl(l, a):
                a = a + hist_v[pl.ds(l * 256 + ci * 16, 16)]
                hist_v[pl.ds(l * 256 + ci * 16, 16)] = jnp.zeros(
                    (16,), jnp.int32)
                return a
            acc = lax.fori_loop(0, 16, fl, jnp.zeros((16,), jnp.int32), unroll=4)
            plsc.store_scatter(tbuf_v, [lane * 16 + ci], acc)
            return 0
        lax.fori_loop(0, 16, fc, 0)

        def gsum(j, a):
            return a + tbuf_v[pl.ds(j * 16, 16)]
        g = lax.fori_loop(0, 16, gsum, jnp.zeros((16,), jnp.int32))
        sg = lax.rev(plsc.cumsum(lax.rev(g, (0,))), (0,)) - g
        mg = (sg < k_rem) & (sg + g >= k_rem)
        ci = jnp.sum(jnp.where(mg, lane, 0))
        s_chunk = jnp.sum(jnp.where(mg, sg, 0))
        t = plsc.load_gather(tbuf_v, [lane * 16 + ci])
        st = lax.rev(plsc.cumsum(lax.rev(t, (0,))), (0,)) - t + s_chunk
        mt = (st < k_rem) & (st + t >= k_rem)
        j = jnp.sum(jnp.where(mt, lane, 0))
        s_above = jnp.sum(jnp.where(mt, st, 0))
        return ci * 16 + j, k_rem - s_above

    zero_hist()

    def r1(i, _):
        f = row_v[pl.ds(i * 16, 16)]
        key = _sc_key(f)
        keys_v[pl.ds(i * 16, 16)] = key
        d = ((key >> 24) & 255).astype(jnp.int32)
        plsc.addupdate_scatter(hist_v, [lane_base + d], ones16)
        return 0
    lax.fori_loop(0, NV, r1, 0, unroll=8)
    bin0, k_rem = fold_and_pick(jnp.int32(k))
    prefix = bin0.astype(jnp.uint32) << 24

    p24 = prefix >> 24

    def cpass(i, off):
        key = keys_v[pl.ds(i * 16, 16)]
        m2 = (key >> 24) == p24
        plsc.store_compressed(ckey_v.at[pl.ds(off, 16)], key, mask=m2)
        plsc.store_compressed(cidx_v.at[pl.ds(off, 16)], i * 16 + lane,
                              mask=m2)
        return off + jnp.sum(m2.astype(jnp.int32))
    n1 = lax.fori_loop(0, NV, cpass, jnp.int32(0), unroll=8)
    nvc = (n1 + 15) >> 4

    for shift in (16, 8, 0):
        hs = shift + 8

        def rb(i, _, hs=hs, shift=shift, prefix=prefix):
            key = ckey_v[pl.ds(i * 16, 16)]
            act = ((key >> hs) == (prefix >> hs)) & ((i * 16 + lane) < n1)
            d = ((key >> shift) & 255).astype(jnp.int32)
            plsc.addupdate_scatter(hist_v, [lane_base + d], ones16, mask=act)
            return 0
        lax.fori_loop(0, nvc, rb, 0)
        bin_, k_rem = fold_and_pick(k_rem)
        prefix = prefix | (bin_.astype(jnp.uint32) << shift)

    tkey = prefix
    allowed = k_rem

    tki = lax.bitcast_convert_type(jnp.zeros((16,), jnp.uint32) + tkey,
                                   jnp.int32)
    plsc.store_scatter(
        pbuf_v, [lane],
        jnp.where(lane == 0, tki, allowed), mask=lane < 2)

    def tb(blk, run):
        plsc.store_scatter(
            pbuf_v, [jnp.full((16,), 2, jnp.int32) + blk],
            jnp.full((16,), 1, jnp.int32) * run, mask=lane == 0)

        def ti(i, r):
            key = keys_v[pl.ds((blk * NVB + i) * 16, 16)]
            return r + jnp.sum((key == tkey).astype(jnp.int32))
        return lax.fori_loop(0, NVB, ti, run, unroll=8)
    lax.fori_loop(0, NB, tb, jnp.int32(0))

    @pl.when(c == 0)
    def _():
        pltpu.sync_copy(pbuf_v, params.at[s])


def _sc_assign_body(logits, params, eout, wout,
                    col_v, ptab_v, oute_v, outw_v, sem,
                    *, E, T, k, TB):
    c = lax.axis_index("c")
    s = lax.axis_index("s")
    wid = c * 16 + s
    base = wid * TB
    NVB = TB // 16
    lane = lax.iota(jnp.int32, 16)

    col_cp = pltpu.async_copy(logits.at[:, pl.ds(base, TB)], col_v, sem)
    pltpu.sync_copy(params, ptab_v)
    col_cp.wait()

    def _lane0(v):
        return jnp.sum(jnp.where(lane == 0, v, 0))

    te, alw, bpre = [], [], []
    widv = jnp.zeros((16,), jnp.int32) + (2 + wid)
    for e in range(E):
        row = ptab_v[e, pl.ds(0, 16)]
        rowu = lax.bitcast_convert_type(row, jnp.uint32)
        te.append(jnp.sum(jnp.where(lane == 0, rowu, jnp.uint32(0))))
        alw.append(jnp.sum(jnp.where(lane == 1, row, 0)))
        bpre.append(_lane0(plsc.load_gather(
            ptab_v, [jnp.full((16,), e, jnp.int32), widv])))
    minus_inf = jnp.full((16,), -jnp.inf, jnp.float32)

    def p3(i, runs):
        ls = [col_v[e, pl.ds(i * 16, 16)] for e in range(E)]
        m = minus_inf
        for e in range(E):
            m = jnp.maximum(m, ls[e])
        den = jnp.zeros((16,), jnp.float32)
        for e in range(E):
            den = den + jnp.exp(ls[e] - m)
        estar = jnp.full((16,), -1, jnp.int32)
        val = jnp.zeros((16,), jnp.float32)
        new_runs = []
        for e in range(E):
            f = ls[e]
            key = _sc_key(f)
            eqm = key == te[e]
            ic = plsc.cumsum(eqm.astype(jnp.int32))
            sel = (key > te[e]) | (eqm & ((bpre[e] + runs[e] + ic) <= alw[e]))
            estar = jnp.where(sel, jnp.int32(e), estar)
            val = jnp.where(sel, f, val)
            new_runs.append(runs[e] + jnp.sum(jnp.where(lane == 15, ic, 0)))
        picked = estar >= 0
        w = jnp.exp(val - m) / den
        oute_v[pl.ds(i * 16, 16)] = jnp.maximum(estar, jnp.int32(0))
        outw_v[pl.ds(i * 16, 16)] = jnp.where(picked, w, jnp.float32(1.0))
        return tuple(new_runs)
    lax.fori_loop(0, NVB, p3, tuple(jnp.int32(0) for _ in range(E)))

    pltpu.sync_copy(oute_v, eout.at[pl.ds(base, TB)])
    pltpu.sync_copy(outw_v, wout.at[pl.ds(base, TB)])


def _sc_select(logits, *, E, T, k):
    NB = 32
    TB = T // NB
    mesh = plsc.VectorSubcoreMesh(core_axis_name="c", subcore_axis_name="s")
    cp = pltpu.CompilerParams(needs_layout_passes=False)

    thresh = pl.kernel(
        functools.partial(_sc_thresh_body, E=E, T=T, k=k, TB=TB),
        out_type=jax.ShapeDtypeStruct((E, _PW), jnp.int32),
        mesh=mesh,
        scratch_types=[
            pltpu.VMEM((T,), jnp.float32),
            pltpu.VMEM((T,), jnp.uint32),
            pltpu.VMEM((16 * 256,), jnp.int32),
            pltpu.VMEM((256,), jnp.int32),
            pltpu.VMEM((_PW,), jnp.int32),
            pltpu.VMEM((T + 32,), jnp.uint32),
            pltpu.VMEM((T + 32,), jnp.int32),
        ],
        compiler_params=cp,
    )
    params = thresh(logits)

    assign = pl.kernel(
        functools.partial(_sc_assign_body, E=E, T=T, k=k, TB=TB),
        out_type=(
            jax.ShapeDtypeStruct((T,), jnp.int32),
            jax.ShapeDtypeStruct((T,), jnp.float32),
        ),
        mesh=mesh,
        scratch_types=[
            pltpu.VMEM((E, TB), jnp.float32),
            pltpu.VMEM((E, _PW), jnp.int32),
            pltpu.VMEM((TB,), jnp.int32),
            pltpu.VMEM((TB,), jnp.float32),
            pltpu.SemaphoreType.DMA,
        ],
        compiler_params=cp,
    )
    return assign(logits, params)


def kernel(x, W_gate):
    B, S, D = x.shape
    E = W_gate.shape[0]
    T = B * S
    k = min(int(T / E * 1.0), T)

    xf = x.reshape(T, D)
    TBLK = 512
    logits = pl.pallas_call(
        _matmul_body,
        grid=(T // TBLK,),
        in_specs=[
            pl.BlockSpec((E, D), lambda i: (0, 0)),
            pl.BlockSpec((TBLK, D), lambda i: (i, 0)),
        ],
        out_specs=pl.BlockSpec((E, TBLK), lambda i: (0, i)),
        out_shape=jax.ShapeDtypeStruct((E, T), jnp.float32),
    )(W_gate, xf)

    expert, weight = _sc_select(logits, E=E, T=T, k=k)

    expert_out = expert.reshape(B, S, 1)
    weight_out = weight.reshape(B, S, 1).astype(x.dtype)
    aux_loss = jnp.array(0.0, dtype=x.dtype)
    return (expert_out, weight_out, aux_loss)


# speedup vs baseline: 1.1452x; 1.0187x over previous
import functools

import jax
import jax.numpy as jnp
from jax import lax
from jax.experimental import pallas as pl
from jax.experimental.pallas import tpu as pltpu
from jax.experimental.pallas import tpu_sc as plsc

_PW = 48


def _matmul_body(w_ref, x_ref, out_ref):
    out_ref[...] = lax.dot_general(
        w_ref[...], x_ref[...],
        dimension_numbers=(((1,), (1,)), ((), ())),
        preferred_element_type=jnp.float32,
    )


def _sc_key(f):
    bu = lax.bitcast_convert_type(f, jnp.uint32)
    key = jnp.where((bu >> 31) == 0, bu | jnp.uint32(0x80000000), ~bu)
    return jnp.where(f == 0.0, jnp.uint32(0x80000000), key)


def _sc_thresh_body(logits, params, row_v, keys_v, hist_v, tbuf_v, pbuf_v,
                    ckey_v, cidx_v, tix_v, *, E, T, k, TB):
    c = lax.axis_index("c")
    s = lax.axis_index("s")
    NV = T // 16
    NVB = TB // 16
    NB = T // TB
    lane = lax.iota(jnp.int32, 16)
    lane_base = lane * 256
    ones16 = jnp.ones((16,), jnp.int32)

    pltpu.sync_copy(logits.at[s], row_v)

    def zero_hist():
        def zh(j, _):
            hist_v[pl.ds(j * 16, 16)] = jnp.zeros((16,), jnp.int32)
            return 0
        lax.fori_loop(0, (16 * 256) // 16, zh, 0, unroll=8)

    def fold_and_pick(k_rem):
        def fc(ci, _):
            def fl(l, a):
                a = a + hist_v[pl.ds(l * 256 + ci * 16, 16)]
                hist_v[pl.ds(l * 256 + ci * 16, 16)] = jnp.zeros(
                    (16,), jnp.int32)
                return a
            acc = lax.fori_loop(0, 16, fl, jnp.zeros((16,), jnp.int32), unroll=4)
            plsc.store_scatter(tbuf_v, [lane * 16 + ci], acc)
            return 0
        lax.fori_loop(0, 16, fc, 0)

        def gsum(j, a):
            return a + tbuf_v[pl.ds(j * 16, 16)]
        g = lax.fori_loop(0, 16, gsum, jnp.zeros((16,), jnp.int32))
        sg = lax.rev(plsc.cumsum(lax.rev(g, (0,))), (0,)) - g
        mg = (sg < k_rem) & (sg + g >= k_rem)
        ci = jnp.sum(jnp.where(mg, lane, 0))
        s_chunk = jnp.sum(jnp.where(mg, sg, 0))
        t = plsc.load_gather(tbuf_v, [lane * 16 + ci])
        st = lax.rev(plsc.cumsum(lax.rev(t, (0,))), (0,)) - t + s_chunk
        mt = (st < k_rem) & (st + t >= k_rem)
        j = jnp.sum(jnp.where(mt, lane, 0))
        s_above = jnp.sum(jnp.where(mt, st, 0))
        return ci * 16 + j, k_rem - s_above

    zero_hist()

    def r1(i, _):
        f = row_v[pl.ds(i * 16, 16)]
        key = _sc_key(f)
        keys_v[pl.ds(i * 16, 16)] = key
        d = ((key >> 24) & 255).astype(jnp.int32)
        plsc.addupdate_scatter(hist_v, [lane_base + d], ones16)
        return 0
    lax.fori_loop(0, NV, r1, 0, unroll=8)
    bin0, k_rem = fold_and_pick(jnp.int32(k))
    prefix = bin0.astype(jnp.uint32) << 24

    p24 = prefix >> 24

    def cpass(i, off):
        key = keys_v[pl.ds(i * 16, 16)]
        m2 = (key >> 24) == p24
        plsc.store_compressed(ckey_v.at[pl.ds(off, 16)], key, mask=m2)
        plsc.store_compressed(cidx_v.at[pl.ds(off, 16)], i * 16 + lane,
                              mask=m2)
        return off + jnp.sum(m2.astype(jnp.int32))
    n1 = lax.fori_loop(0, NV, cpass, jnp.int32(0), unroll=8)
    nvc = (n1 + 15) >> 4

    for shift in (16, 8, 0):
        hs = shift + 8

        def rb(i, _, hs=hs, shift=shift, prefix=prefix):
            key = ckey_v[pl.ds(i * 16, 16)]
            act = ((key >> hs) == (prefix >> hs)) & ((i * 16 + lane) < n1)
            d = ((key >> shift) & 255).astype(jnp.int32)
            plsc.addupdate_scatter(hist_v, [lane_base + d], ones16, mask=act)
            return 0
        lax.fori_loop(0, nvc, rb, 0)
        bin_, k_rem = fold_and_pick(k_rem)
        prefix = prefix | (bin_.astype(jnp.uint32) << shift)

    tkey = prefix
    allowed = k_rem

    tki = lax.bitcast_convert_type(jnp.zeros((16,), jnp.uint32) + tkey,
                                   jnp.int32)
    plsc.store_scatter(
        pbuf_v, [lane],
        jnp.where(lane == 0, tki, allowed), mask=lane < 2)

    def tpass(i, off):
        key = ckey_v[pl.ds(i * 16, 16)]
        idxv = cidx_v[pl.ds(i * 16, 16)]
        m = (key == tkey) & ((i * 16 + lane) < n1)
        plsc.store_compressed(tix_v.at[pl.ds(off, 16)], idxv, mask=m)
        return off + jnp.sum(m.astype(jnp.int32))
    nt = lax.fori_loop(0, nvc, tpass, jnp.int32(0))
    ntv = (nt + 15) >> 4

    def tb(blk, _):
        def ti(i, r):
            idxv = tix_v[pl.ds(i * 16, 16)]
            mm = (idxv < blk * TB) & ((i * 16 + lane) < nt)
            return r + jnp.sum(mm.astype(jnp.int32))
        run = lax.fori_loop(0, ntv, ti, jnp.int32(0))
        plsc.store_scatter(
            pbuf_v, [jnp.full((16,), 2, jnp.int32) + blk],
            jnp.full((16,), 1, jnp.int32) * run, mask=lane == 0)
        return 0
    lax.fori_loop(0, NB, tb, 0)

    @pl.when(c == 0)
    def _():
        pltpu.sync_copy(pbuf_v, params.at[s])


def _sc_assign_body(logits, params, eout, wout,
                    col_v, ptab_v, oute_v, outw_v, sem,
                    *, E, T, k, TB):
    c = lax.axis_index("c")
    s = lax.axis_index("s")
    wid = c * 16 + s
    base = wid * TB
    NVB = TB // 16
    lane = lax.iota(jnp.int32, 16)

    col_cp = pltpu.async_copy(logits.at[:, pl.ds(base, TB)], col_v, sem)
    pltpu.sync_copy(params, ptab_v)
    col_cp.wait()

    def _lane0(v):
        return jnp.sum(jnp.where(lane == 0, v, 0))

    te, alw, bpre = [], [], []
    widv = jnp.zeros((16,), jnp.int32) + (2 + wid)
    for e in range(E):
        row = ptab_v[e, pl.ds(0, 16)]
        rowu = lax.bitcast_convert_type(row, jnp.uint32)
        te.append(jnp.sum(jnp.where(lane == 0, rowu, jnp.uint32(0))))
        alw.append(jnp.sum(jnp.where(lane == 1, row, 0)))
        bpre.append(_lane0(plsc.load_gather(
            ptab_v, [jnp.full((16,), e, jnp.int32), widv])))
    minus_inf = jnp.full((16,), -jnp.inf, jnp.float32)

    def p3(i, runs):
        ls = [col_v[e, pl.ds(i * 16, 16)] for e in range(E)]
        m = minus_inf
        for e in range(E):
            m = jnp.maximum(m, ls[e])
        den = jnp.zeros((16,), jnp.float32)
        for e in range(E):
            den = den + jnp.exp(ls[e] - m)
        estar = jnp.full((16,), -1, jnp.int32)
        val = jnp.zeros((16,), jnp.float32)
        new_runs = []
        for e in range(E):
            f = ls[e]
            key = _sc_key(f)
            eqm = key == te[e]
            ic = plsc.cumsum(eqm.astype(jnp.int32))
            sel = (key > te[e]) | (eqm & ((bpre[e] + runs[e] + ic) <= alw[e]))
            estar = jnp.where(sel, jnp.int32(e), estar)
            val = jnp.where(sel, f, val)
            new_runs.append(runs[e] + jnp.sum(jnp.where(lane == 15, ic, 0)))
        picked = estar >= 0
        w = jnp.exp(val - m) / den
        oute_v[pl.ds(i * 16, 16)] = jnp.maximum(estar, jnp.int32(0))
        outw_v[pl.ds(i * 16, 16)] = jnp.where(picked, w, jnp.float32(1.0))
        return tuple(new_runs)
    lax.fori_loop(0, NVB, p3, tuple(jnp.int32(0) for _ in range(E)))

    pltpu.sync_copy(oute_v, eout.at[pl.ds(base, TB)])
    pltpu.sync_copy(outw_v, wout.at[pl.ds(base, TB)])


def _sc_select(logits, *, E, T, k):
    NB = 32
    TB = T // NB
    mesh = plsc.VectorSubcoreMesh(core_axis_name="c", subcore_axis_name="s")
    cp = pltpu.CompilerParams(needs_layout_passes=False)

    thresh = pl.kernel(
        functools.partial(_sc_thresh_body, E=E, T=T, k=k, TB=TB),
        out_type=jax.ShapeDtypeStruct((E, _PW), jnp.int32),
        mesh=mesh,
        scratch_types=[
            pltpu.VMEM((T,), jnp.float32),
            pltpu.VMEM((T,), jnp.uint32),
            pltpu.VMEM((16 * 256,), jnp.int32),
            pltpu.VMEM((256,), jnp.int32),
            pltpu.VMEM((_PW,), jnp.int32),
            pltpu.VMEM((T + 32,), jnp.uint32),
            pltpu.VMEM((T + 32,), jnp.int32),
            pltpu.VMEM((T + 32,), jnp.int32),
        ],
        compiler_params=cp,
    )
    params = thresh(logits)

    assign = pl.kernel(
        functools.partial(_sc_assign_body, E=E, T=T, k=k, TB=TB),
        out_type=(
            jax.ShapeDtypeStruct((T,), jnp.int32),
            jax.ShapeDtypeStruct((T,), jnp.float32),
        ),
        mesh=mesh,
        scratch_types=[
            pltpu.VMEM((E, TB), jnp.float32),
            pltpu.VMEM((E, _PW), jnp.int32),
            pltpu.VMEM((TB,), jnp.int32),
            pltpu.VMEM((TB,), jnp.float32),
            pltpu.SemaphoreType.DMA,
        ],
        compiler_params=cp,
    )
    return assign(logits, params)


def kernel(x, W_gate):
    B, S, D = x.shape
    E = W_gate.shape[0]
    T = B * S
    k = min(int(T / E * 1.0), T)

    xf = x.reshape(T, D)
    TBLK = 1024
    logits = pl.pallas_call(
        _matmul_body,
        grid=(T // TBLK,),
        in_specs=[
            pl.BlockSpec((E, D), lambda i: (0, 0)),
            pl.BlockSpec((TBLK, D), lambda i: (i, 0)),
        ],
        out_specs=pl.BlockSpec((E, TBLK), lambda i: (0, i)),
        out_shape=jax.ShapeDtypeStruct((E, T), jnp.float32),
    )(W_gate, xf)

    expert, weight = _sc_select(logits, E=E, T=T, k=k)

    expert_out = expert.reshape(B, S, 1)
    weight_out = weight.reshape(B, S, 1).astype(x.dtype)
    aux_loss = jnp.array(0.0, dtype=x.dtype)
    return (expert_out, weight_out, aux_loss)


# speedup vs baseline: 1.1454x; 1.0002x over previous
import functools

import jax
import jax.numpy as jnp
from jax import lax
from jax.experimental import pallas as pl
from jax.experimental.pallas import tpu as pltpu
from jax.experimental.pallas import tpu_sc as plsc

_PW = 48


def _matmul_body(w_ref, x_ref, out_ref):
    h = x_ref.shape[0] // 2
    dn = (((1,), (1,)), ((), ()))
    out_ref[:, :h] = lax.dot_general(
        w_ref[...], x_ref[:h, :], dimension_numbers=dn,
        preferred_element_type=jnp.float32)
    out_ref[:, h:] = lax.dot_general(
        w_ref[...], x_ref[h:, :], dimension_numbers=dn,
        preferred_element_type=jnp.float32)


def _sc_key(f):
    bu = lax.bitcast_convert_type(f, jnp.uint32)
    key = jnp.where((bu >> 31) == 0, bu | jnp.uint32(0x80000000), ~bu)
    return jnp.where(f == 0.0, jnp.uint32(0x80000000), key)


def _sc_thresh_body(logits, params, row_v, keys_v, hist_v, tbuf_v, pbuf_v,
                    ckey_v, cidx_v, tix_v, *, E, T, k, TB):
    c = lax.axis_index("c")
    s = lax.axis_index("s")
    NV = T // 16
    NVB = TB // 16
    NB = T // TB
    lane = lax.iota(jnp.int32, 16)
    lane_base = lane * 256
    ones16 = jnp.ones((16,), jnp.int32)

    pltpu.sync_copy(logits.at[s], row_v)

    def zero_hist():
        def zh(j, _):
            hist_v[pl.ds(j * 16, 16)] = jnp.zeros((16,), jnp.int32)
            return 0
        lax.fori_loop(0, (16 * 256) // 16, zh, 0, unroll=8)

    def fold_and_pick(k_rem):
        def fc(ci, _):
            def fl(l, a):
                a = a + hist_v[pl.ds(l * 256 + ci * 16, 16)]
                hist_v[pl.ds(l * 256 + ci * 16, 16)] = jnp.zeros(
                    (16,), jnp.int32)
                return a
            acc = lax.fori_loop(0, 16, fl, jnp.zeros((16,), jnp.int32), unroll=4)
            plsc.store_scatter(tbuf_v, [lane * 16 + ci], acc)
            return 0
        lax.fori_loop(0, 16, fc, 0)

        def gsum(j, a):
            return a + tbuf_v[pl.ds(j * 16, 16)]
        g = lax.fori_loop(0, 16, gsum, jnp.zeros((16,), jnp.int32))
        sg = lax.rev(plsc.cumsum(lax.rev(g, (0,))), (0,)) - g
        mg = (sg < k_rem) & (sg + g >= k_rem)
        ci = jnp.sum(jnp.where(mg, lane, 0))
        s_chunk = jnp.sum(jnp.where(mg, sg, 0))
        t = plsc.load_gather(tbuf_v, [lane * 16 + ci])
        st = lax.rev(plsc.cumsum(lax.rev(t, (0,))), (0,)) - t + s_chunk
        mt = (st < k_rem) & (st + t >= k_rem)
        j = jnp.sum(jnp.where(mt, lane, 0))
        s_above = jnp.sum(jnp.where(mt, st, 0))
        return ci * 16 + j, k_rem - s_above

    zero_hist()

    def r1(i, _):
        f = row_v[pl.ds(i * 16, 16)]
        key = _sc_key(f)
        keys_v[pl.ds(i * 16, 16)] = key
        d = ((key >> 24) & 255).astype(jnp.int32)
        plsc.addupdate_scatter(hist_v, [lane_base + d], ones16)
        return 0
    lax.fori_loop(0, NV, r1, 0, unroll=8)
    bin0, k_rem = fold_and_pick(jnp.int32(k))
    prefix = bin0.astype(jnp.uint32) << 24

    p24 = prefix >> 24

    def cpass(i, off):
        key = keys_v[pl.ds(i * 16, 16)]
        m2 = (key >> 24) == p24
        plsc.store_compressed(ckey_v.at[pl.ds(off, 16)], key, mask=m2)
        plsc.store_compressed(cidx_v.at[pl.ds(off, 16)], i * 16 + lane,
                              mask=m2)
        return off + jnp.sum(m2.astype(jnp.int32))
    n1 = lax.fori_loop(0, NV, cpass, jnp.int32(0), unroll=8)
    nvc = (n1 + 15) >> 4

    for shift in (16, 8, 0):
        hs = shift + 8

        def rb(i, _, hs=hs, shift=shift, prefix=prefix):
            key = ckey_v[pl.ds(i * 16, 16)]
            act = ((key >> hs) == (prefix >> hs)) & ((i * 16 + lane) < n1)
            d = ((key >> shift) & 255).astype(jnp.int32)
            plsc.addupdate_scatter(hist_v, [lane_base + d], ones16, mask=act)
            return 0
        lax.fori_loop(0, nvc, rb, 0)
        bin_, k_rem = fold_and_pick(k_rem)
        prefix = prefix | (bin_.astype(jnp.uint32) << shift)

    tkey = prefix
    allowed = k_rem

    tki = lax.bitcast_convert_type(jnp.zeros((16,), jnp.uint32) + tkey,
                                   jnp.int32)
    plsc.store_scatter(
        pbuf_v, [lane],
        jnp.where(lane == 0, tki, allowed), mask=lane < 2)

    def tpass(i, off):
        key = ckey_v[pl.ds(i * 16, 16)]
        idxv = cidx_v[pl.ds(i * 16, 16)]
        m = (key == tkey) & ((i * 16 + lane) < n1)
        plsc.store_compressed(tix_v.at[pl.ds(off, 16)], idxv, mask=m)
        return off + jnp.sum(m.astype(jnp.int32))
    nt = lax.fori_loop(0, nvc, tpass, jnp.int32(0))
    ntv = (nt + 15) >> 4

    def tb(blk, _):
        def ti(i, r):
            idxv = tix_v[pl.ds(i * 16, 16)]
            mm = (idxv < blk * TB) & ((i * 16 + lane) < nt)
            return r + jnp.sum(mm.astype(jnp.int32))
        run = lax.fori_loop(0, ntv, ti, jnp.int32(0))
        plsc.store_scatter(
            pbuf_v, [jnp.full((16,), 2, jnp.int32) + blk],
            jnp.full((16,), 1, jnp.int32) * run, mask=lane == 0)
        return 0
    lax.fori_loop(0, NB, tb, 0)

    @pl.when(c == 0)
    def _():
        pltpu.sync_copy(pbuf_v, params.at[s])


def _sc_assign_body(logits, params, eout, wout,
                    col_v, ptab_v, oute_v, outw_v, sem,
                    *, E, T, k, TB):
    c = lax.axis_index("c")
    s = lax.axis_index("s")
    wid = c * 16 + s
    base = wid * TB
    NVB = TB // 16
    lane = lax.iota(jnp.int32, 16)

    col_cp = pltpu.async_copy(logits.at[:, pl.ds(base, TB)], col_v, sem)
    pltpu.sync_copy(params, ptab_v)
    col_cp.wait()

    def _lane0(v):
        return jnp.sum(jnp.where(lane == 0, v, 0))

    te, alw, bpre = [], [], []
    widv = jnp.zeros((16,), jnp.int32) + (2 + wid)
    for e in range(E):
        row = ptab_v[e, pl.ds(0, 16)]
        rowu = lax.bitcast_convert_type(row, jnp.uint32)
        te.append(jnp.sum(jnp.where(lane == 0, rowu, jnp.uint32(0))))
        alw.append(jnp.sum(jnp.where(lane == 1, row, 0)))
        bpre.append(_lane0(plsc.load_gather(
            ptab_v, [jnp.full((16,), e, jnp.int32), widv])))
    minus_inf = jnp.full((16,), -jnp.inf, jnp.float32)

    def p3(i, runs):
        ls = [col_v[e, pl.ds(i * 16, 16)] for e in range(E)]
        m = minus_inf
        for e in range(E):
            m = jnp.maximum(m, ls[e])
        den = jnp.zeros((16,), jnp.float32)
        for e in range(E):
            den = den + jnp.exp(ls[e] - m)
        estar = jnp.full((16,), -1, jnp.int32)
        val = jnp.zeros((16,), jnp.float32)
        new_runs = []
        for e in range(E):
            f = ls[e]
            key = _sc_key(f)
            eqm = key == te[e]
            ic = plsc.cumsum(eqm.astype(jnp.int32))
            sel = (key > te[e]) | (eqm & ((bpre[e] + runs[e] + ic) <= alw[e]))
            estar = jnp.where(sel, jnp.int32(e), estar)
            val = jnp.where(sel, f, val)
            new_runs.append(runs[e] + jnp.sum(jnp.where(lane == 15, ic, 0)))
        picked = estar >= 0
        w = jnp.exp(val - m) / den
        oute_v[pl.ds(i * 16, 16)] = jnp.maximum(estar, jnp.int32(0))
        outw_v[pl.ds(i * 16, 16)] = jnp.where(picked, w, jnp.float32(1.0))
        return tuple(new_runs)
    lax.fori_loop(0, NVB, p3, tuple(jnp.int32(0) for _ in range(E)))

    pltpu.sync_copy(oute_v, eout.at[pl.ds(base, TB)])
    pltpu.sync_copy(outw_v, wout.at[pl.ds(base, TB)])


def _sc_select(logits, *, E, T, k):
    NB = 32
    TB = T // NB
    mesh = plsc.VectorSubcoreMesh(core_axis_name="c", subcore_axis_name="s")
    cp = pltpu.CompilerParams(needs_layout_passes=False)

    thresh = pl.kernel(
        functools.partial(_sc_thresh_body, E=E, T=T, k=k, TB=TB),
        out_type=jax.ShapeDtypeStruct((E, _PW), jnp.int32),
        mesh=mesh,
        scratch_types=[
            pltpu.VMEM((T,), jnp.float32),
            pltpu.VMEM((T,), jnp.uint32),
            pltpu.VMEM((16 * 256,), jnp.int32),
            pltpu.VMEM((256,), jnp.int32),
            pltpu.VMEM((_PW,), jnp.int32),
            pltpu.VMEM((T + 32,), jnp.uint32),
            pltpu.VMEM((T + 32,), jnp.int32),
            pltpu.VMEM((T + 32,), jnp.int32),
        ],
        compiler_params=cp,
    )
    params = thresh(logits)

    assign = pl.kernel(
        functools.partial(_sc_assign_body, E=E, T=T, k=k, TB=TB),
        out_type=(
            jax.ShapeDtypeStruct((T,), jnp.int32),
            jax.ShapeDtypeStruct((T,), jnp.float32),
        ),
        mesh=mesh,
        scratch_types=[
            pltpu.VMEM((E, TB), jnp.float32),
            pltpu.VMEM((E, _PW), jnp.int32),
            pltpu.VMEM((TB,), jnp.int32),
            pltpu.VMEM((TB,), jnp.float32),
            pltpu.SemaphoreType.DMA,
        ],
        compiler_params=cp,
    )
    return assign(logits, params)


def kernel(x, W_gate):
    B, S, D = x.shape
    E = W_gate.shape[0]
    T = B * S
    k = min(int(T / E * 1.0), T)

    xf = x.reshape(T, D)
    TBLK = 1024
    logits = pl.pallas_call(
        _matmul_body,
        grid=(T // TBLK,),
        in_specs=[
            pl.BlockSpec((E, D), lambda i: (0, 0)),
            pl.BlockSpec((TBLK, D), lambda i: (i, 0)),
        ],
        out_specs=pl.BlockSpec((E, TBLK), lambda i: (0, i)),
        out_shape=jax.ShapeDtypeStruct((E, T), jnp.float32),
    )(W_gate, xf)

    expert, weight = _sc_select(logits, E=E, T=T, k=k)

    expert_out = expert.reshape(B, S, 1)
    weight_out = weight.reshape(B, S, 1).astype(x.dtype)
    aux_loss = jnp.array(0.0, dtype=x.dtype)
    return (expert_out, weight_out, aux_loss)
